# Initial kernel scaffold; baseline (speedup 1.0000x reference)
import jax
import jax.numpy as jnp
from jax.experimental import pallas as pl


def kernel(x, edge_index, batch, W1, b1, W2, b2, W3, b3, Wr1, br1, Wr2, br2):
    raise NotImplementedError("write your pallas kernel here")


# speedup vs baseline: 8.2905x; 8.2905x over previous
import functools

import jax
import jax.numpy as jnp
from jax import lax
from jax.experimental import pallas as pl
from jax.experimental.pallas import tpu as pltpu
from jax.experimental.pallas import tpu_sc as plsc

N = 10000
E = 320000
D = 128
NG = 64
NC = 2
NS = 16
NW = NC * NS
CH = 128
K = -(-E // (NW * CH))
EPAD = NW * CH * K
RPT = 632
NACC = NS * RPT


def _sc_mesh():
    return plsc.VectorSubcoreMesh(
        core_axis_name="c", subcore_axis_name="s", num_cores=NC, num_subcores=NS
    )


def _scatter_body(g_hbm, srcl, dstl, zrows, out_hbm, sidx, didx, rows, acc, sem):
    c = lax.axis_index("c")
    s = lax.axis_index("s")
    w = c * NS + s
    pltpu.sync_copy(zrows, acc.at[pl.ds(s * RPT, RPT)])
    pltpu.sync_copy(srcl.at[w], sidx)
    pltpu.sync_copy(dstl.at[w], didx)
    plsc.subcore_barrier()

    def step(k, carry):
        pltpu.async_copy(g_hbm.at[sidx.at[k]], rows, sem).wait()
        pltpu.sync_copy(rows, acc.at[didx.at[k]], add=True)
        return carry

    lax.fori_loop(0, K, step, 0)
    plsc.subcore_barrier()
    pltpu.sync_copy(acc.at[pl.ds(s * RPT, RPT)], out_hbm.at[c, pl.ds(s * RPT, RPT)])


def _edge_scatter(g, srcl, dstl, zrows):
    return pl.kernel(
        _scatter_body,
        out_type=jax.ShapeDtypeStruct((NC, NACC, D), jnp.float32),
        mesh=_sc_mesh(),
        scratch_types=[
            pltpu.VMEM((K, CH), jnp.int32),
            pltpu.VMEM((K, CH), jnp.int32),
            pltpu.VMEM((CH, D), jnp.float32),
            pltpu.VMEM_SHARED((NACC, D), jnp.float32),
            pltpu.SemaphoreType.DMA,
        ],
    )(g, srcl, dstl, zrows)


def _dinv_from(pdeg_ref):
    d = pdeg_ref[0][:N, 0:1] + pdeg_ref[1][:N, 0:1]
    return lax.rsqrt(d + 1.0)


def _k1_body(x_ref, w_ref, pdeg_ref, g_ref):
    dinv = _dinv_from(pdeg_ref)
    h = jnp.dot(x_ref[...], w_ref[...], preferred_element_type=jnp.float32)
    g_ref[...] = h * dinv


def _mid_body(p_ref, g_ref, pdeg_ref, w_ref, b_ref, o_ref):
    dinv = _dinv_from(pdeg_ref)
    h = dinv * (p_ref[0][:N] + p_ref[1][:N] + g_ref[...]) + b_ref[...]
    a = jnp.maximum(h, 0.0)
    o_ref[...] = dinv * jnp.dot(a, w_ref[...], preferred_element_type=jnp.float32)


def _readout_body(p_ref, g_ref, pdeg_ref, b3_ref, wr1_ref, br1_ref, wr2_ref,
                  br2_ref, batch_ref, o_ref):
    dinv = _dinv_from(pdeg_ref)
    h = dinv * (p_ref[0][:N] + p_ref[1][:N] + g_ref[...]) + b3_ref[...]
    t = jnp.maximum(
        jnp.dot(h, wr1_ref[...], preferred_element_type=jnp.float32) + br1_ref[...],
        0.0,
    )
    r = jnp.dot(t, wr2_ref[...], preferred_element_type=jnp.float32) + br2_ref[...]
    onehot = (batch_ref[...] == lax.broadcasted_iota(jnp.int32, (N, NG), 1)
              ).astype(jnp.float32)
    dn = (((0,), (0,)), ((), ()))
    sums = lax.dot_general(onehot, r, dn, preferred_element_type=jnp.float32)
    counts = lax.dot_general(onehot, jnp.ones((N, 1), jnp.float32), dn,
                             preferred_element_type=jnp.float32)
    o_ref[...] = sums / jnp.maximum(counts, 1.0)


def kernel(x, edge_index, batch, W1, b1, W2, b2, W3, b3, Wr1, br1, Wr2, br2):
    src = edge_index[0].astype(jnp.int32)
    dst = edge_index[1].astype(jnp.int32)
    pad = EPAD - E
    srcl = jnp.concatenate([src, jnp.zeros((pad,), jnp.int32)]).reshape(NW, K, CH)
    pad_dst = N + (jnp.arange(pad, dtype=jnp.int32) % 16)
    dstl = jnp.concatenate([dst, pad_dst]).reshape(NW, K, CH)
    zf = jnp.zeros((RPT, D), jnp.float32)

    pdeg = _edge_scatter(jnp.ones((N, D), jnp.float32), srcl, dstl, zf)

    g1 = pl.pallas_call(
        _k1_body, out_shape=jax.ShapeDtypeStruct((N, D), jnp.float32),
    )(x, W1, pdeg)
    p1 = _edge_scatter(g1, srcl, dstl, zf)

    mid = pl.pallas_call(
        _mid_body, out_shape=jax.ShapeDtypeStruct((N, D), jnp.float32),
    )
    g2 = mid(p1, g1, pdeg, W2, b1.reshape(1, D))
    p2 = _edge_scatter(g2, srcl, dstl, zf)

    g3 = mid(p2, g2, pdeg, W3, b2.reshape(1, D))
    p3 = _edge_scatter(g3, srcl, dstl, zf)

    out = pl.pallas_call(
        _readout_body, out_shape=jax.ShapeDtypeStruct((NG, 1), jnp.float32),
    )(p3, g3, pdeg, b3.reshape(1, D), Wr1, br1.reshape(1, D // 2), Wr2,
      br2.reshape(1, 1), batch.astype(jnp.int32).reshape(N, 1))
    return out


# speedup vs baseline: 8.3109x; 1.0025x over previous
import functools

import jax
import jax.numpy as jnp
from jax import lax
from jax.experimental import pallas as pl
from jax.experimental.pallas import tpu as pltpu
from jax.experimental.pallas import tpu_sc as plsc

N = 10000
E = 320000
D = 128
NG = 64
NC = 2
NS = 16
NW = NC * NS
CH = 128
K = -(-E // (NW * CH))
EPAD = NW * CH * K
RPT = 632
NACC = NS * RPT
NT = N + 8


def _sc_mesh():
    return plsc.VectorSubcoreMesh(
        core_axis_name="c", subcore_axis_name="s", num_cores=NC, num_subcores=NS
    )


def _scatter_body(g_hbm, srcl, dstl, zrows, out_hbm, sidx, didx, rows, acc, sem):
    c = lax.axis_index("c")
    s = lax.axis_index("s")
    w = c * NS + s
    pltpu.sync_copy(zrows, acc.at[pl.ds(s * RPT, RPT)])
    pltpu.sync_copy(srcl.at[w], sidx)
    pltpu.sync_copy(dstl.at[w], didx)
    plsc.subcore_barrier()

    def step(k, carry):
        pltpu.async_copy(g_hbm.at[sidx.at[k]], rows, sem).wait()
        pltpu.sync_copy(rows, acc.at[didx.at[k]], add=True)
        return carry

    lax.fori_loop(0, K, step, 0)
    plsc.subcore_barrier()
    pltpu.sync_copy(acc.at[pl.ds(s * RPT, RPT)], out_hbm.at[c, pl.ds(s * RPT, RPT)])


def _edge_scatter(g, srcl, dstl, zrows):
    return pl.kernel(
        _scatter_body,
        out_type=jax.ShapeDtypeStruct((NC, NACC, D), jnp.float32),
        mesh=_sc_mesh(),
        scratch_types=[
            pltpu.VMEM((K, CH), jnp.int32),
            pltpu.VMEM((K, CH), jnp.int32),
            pltpu.VMEM((CH, D), jnp.float32),
            pltpu.VMEM_SHARED((NACC, D), jnp.float32),
            pltpu.SemaphoreType.DMA,
        ],
    )(g, srcl, dstl, zrows)


def _dinv_from(pdeg_ref):
    d = pdeg_ref[0][:N, 0:1] + pdeg_ref[1][:N, 0:1]
    return lax.rsqrt(d + 1.0)


def _k1_body(x_ref, w_ref, pdeg_ref, g_ref):
    dinv = _dinv_from(pdeg_ref)
    h = jnp.dot(x_ref[...], w_ref[...], preferred_element_type=jnp.float32)
    g_ref[pl.ds(0, N)] = h * dinv
    g_ref[pl.ds(N, NT - N)] = jnp.zeros((NT - N, D), jnp.float32)


def _mid_body(p_ref, g_ref, pdeg_ref, w_ref, b_ref, o_ref):
    dinv = _dinv_from(pdeg_ref)
    h = dinv * (p_ref[0][:N] + p_ref[1][:N] + g_ref[:N]) + b_ref[...]
    a = jnp.maximum(h, 0.0)
    o_ref[pl.ds(0, N)] = dinv * jnp.dot(a, w_ref[...], preferred_element_type=jnp.float32)
    o_ref[pl.ds(N, NT - N)] = jnp.zeros((NT - N, D), jnp.float32)


def _readout_body(p_ref, g_ref, pdeg_ref, b3_ref, wr1_ref, br1_ref, wr2_ref,
                  br2_ref, batch_ref, o_ref):
    dinv = _dinv_from(pdeg_ref)
    h = dinv * (p_ref[0][:N] + p_ref[1][:N] + g_ref[:N]) + b3_ref[...]
    t = jnp.maximum(
        jnp.dot(h, wr1_ref[...], preferred_element_type=jnp.float32) + br1_ref[...],
        0.0,
    )
    r = jnp.dot(t, wr2_ref[...], preferred_element_type=jnp.float32) + br2_ref[...]
    onehot = (batch_ref[...] == lax.broadcasted_iota(jnp.int32, (N, NG), 1)
              ).astype(jnp.float32)
    dn = (((0,), (0,)), ((), ()))
    sums = lax.dot_general(onehot, r, dn, preferred_element_type=jnp.float32)
    counts = lax.dot_general(onehot, jnp.ones((N, 1), jnp.float32), dn,
                             preferred_element_type=jnp.float32)
    o_ref[...] = sums / jnp.maximum(counts, 1.0)


def kernel(x, edge_index, batch, W1, b1, W2, b2, W3, b3, Wr1, br1, Wr2, br2):
    src = edge_index[0].astype(jnp.int32)
    dst = edge_index[1].astype(jnp.int32)
    pad = EPAD - E
    srcl = jnp.concatenate([src, jnp.full((pad,), N, jnp.int32)]).reshape(NW, K, CH)
    pad_dst = jnp.arange(pad, dtype=jnp.int32) % N
    dstl = jnp.concatenate([dst, pad_dst]).reshape(NW, K, CH)
    zf = jnp.zeros((RPT, D), jnp.float32)

    onest = jnp.concatenate([jnp.ones((N, D), jnp.float32),
                             jnp.zeros((NT - N, D), jnp.float32)])
    pdeg = _edge_scatter(onest, srcl, dstl, zf)

    g1 = pl.pallas_call(
        _k1_body, out_shape=jax.ShapeDtypeStruct((NT, D), jnp.float32),
    )(x, W1, pdeg)
    p1 = _edge_scatter(g1, srcl, dstl, zf)

    mid = pl.pallas_call(
        _mid_body, out_shape=jax.ShapeDtypeStruct((NT, D), jnp.float32),
    )
    g2 = mid(p1, g1, pdeg, W2, b1.reshape(1, D))
    p2 = _edge_scatter(g2, srcl, dstl, zf)

    g3 = mid(p2, g2, pdeg, W3, b2.reshape(1, D))
    p3 = _edge_scatter(g3, srcl, dstl, zf)

    out = pl.pallas_call(
        _readout_body, out_shape=jax.ShapeDtypeStruct((NG, 1), jnp.float32),
    )(p3, g3, pdeg, b3.reshape(1, D), Wr1, br1.reshape(1, D // 2), Wr2,
      br2.reshape(1, 1), batch.astype(jnp.int32).reshape(N, 1))
    return out


# speedup vs baseline: 10.8541x; 1.3060x over previous
import functools

import jax
import jax.numpy as jnp
from jax import lax
from jax.experimental import pallas as pl
from jax.experimental.pallas import tpu as pltpu
from jax.experimental.pallas import tpu_sc as plsc

N = 10000
E = 320000
D = 128
NG = 64
NC = 2
NS = 16
NW = NC * NS
CH = 128
K = -(-E // (NW * CH))
EPAD = NW * CH * K
RPT = 632
NACC = NS * RPT
NT = N + 8


def _sc_mesh():
    return plsc.VectorSubcoreMesh(
        core_axis_name="c", subcore_axis_name="s", num_cores=NC, num_subcores=NS
    )


def _scatter_body(g_hbm, sdl, zrows, out_hbm, sd0, sd1, rows0, rows1, acc,
                  isem0, isem1, gsem0, gsem1):
    c = lax.axis_index("c")
    s = lax.axis_index("s")
    w = c * NS + s
    pltpu.sync_copy(zrows, acc.at[pl.ds(s * RPT, RPT)])
    plsc.subcore_barrier()

    pltpu.sync_copy(sdl.at[w, 0], sd0)
    pltpu.async_copy(g_hbm.at[sd0.at[0]], rows0, gsem0)
    pltpu.async_copy(sdl.at[w, 1], sd1, isem1)

    def pair(i, carry):
        k0 = i * 2
        pltpu.make_async_copy(sdl.at[w, k0 + 1], sd1, isem1).wait()
        pltpu.async_copy(g_hbm.at[sd1.at[0]], rows1, gsem1)
        pltpu.make_async_copy(g_hbm.at[sd0.at[0]], rows0, gsem0).wait()
        pltpu.sync_copy(rows0, acc.at[sd0.at[1]], add=True)
        pltpu.async_copy(sdl.at[w, k0 + 2], sd0, isem0)
        pltpu.make_async_copy(sdl.at[w, k0 + 2], sd0, isem0).wait()
        pltpu.async_copy(g_hbm.at[sd0.at[0]], rows0, gsem0)
        pltpu.make_async_copy(g_hbm.at[sd1.at[0]], rows1, gsem1).wait()
        pltpu.sync_copy(rows1, acc.at[sd1.at[1]], add=True)
        pltpu.async_copy(sdl.at[w, k0 + 3], sd1, isem1)
        return carry

    lax.fori_loop(0, (K - 3) // 2, pair, 0)
    pltpu.make_async_copy(sdl.at[w, K - 2], sd1, isem1).wait()
    pltpu.async_copy(g_hbm.at[sd1.at[0]], rows1, gsem1)
    pltpu.make_async_copy(g_hbm.at[sd0.at[0]], rows0, gsem0).wait()
    pltpu.sync_copy(rows0, acc.at[sd0.at[1]], add=True)
    pltpu.sync_copy(sdl.at[w, K - 1], sd0)
    pltpu.async_copy(g_hbm.at[sd0.at[0]], rows0, gsem0)
    pltpu.make_async_copy(g_hbm.at[sd1.at[0]], rows1, gsem1).wait()
    pltpu.sync_copy(rows1, acc.at[sd1.at[1]], add=True)
    pltpu.make_async_copy(g_hbm.at[sd0.at[0]], rows0, gsem0).wait()
    pltpu.sync_copy(rows0, acc.at[sd0.at[1]], add=True)
    plsc.subcore_barrier()
    pltpu.sync_copy(acc.at[pl.ds(s * RPT, RPT)], out_hbm.at[c, pl.ds(s * RPT, RPT)])


def _edge_scatter(g, sdl, zrows):
    return pl.kernel(
        _scatter_body,
        out_type=jax.ShapeDtypeStruct((NC, NACC, D), jnp.float32),
        mesh=_sc_mesh(),
        scratch_types=[
            pltpu.VMEM((2, CH), jnp.int32),
            pltpu.VMEM((2, CH), jnp.int32),
            pltpu.VMEM((CH, D), jnp.float32),
            pltpu.VMEM((CH, D), jnp.float32),
            pltpu.VMEM_SHARED((NACC, D), jnp.float32),
            pltpu.SemaphoreType.DMA,
            pltpu.SemaphoreType.DMA,
            pltpu.SemaphoreType.DMA,
            pltpu.SemaphoreType.DMA,
        ],
    )(g, sdl, zrows)


def _dinv_from(pdeg_ref):
    d = pdeg_ref[0][:N, 0:1] + pdeg_ref[1][:N, 0:1]
    return lax.rsqrt(d + 1.0)


def _k1_body(x_ref, w_ref, pdeg_ref, g_ref):
    dinv = _dinv_from(pdeg_ref)
    h = jnp.dot(x_ref[...], w_ref[...], preferred_element_type=jnp.float32)
    g_ref[pl.ds(0, N)] = h * dinv
    g_ref[pl.ds(N, NT - N)] = jnp.zeros((NT - N, D), jnp.float32)


def _mid_body(p_ref, g_ref, pdeg_ref, w_ref, b_ref, o_ref):
    dinv = _dinv_from(pdeg_ref)
    h = dinv * (p_ref[0][:N] + p_ref[1][:N] + g_ref[:N]) + b_ref[...]
    a = jnp.maximum(h, 0.0)
    o_ref[pl.ds(0, N)] = dinv * jnp.dot(a, w_ref[...], preferred_element_type=jnp.float32)
    o_ref[pl.ds(N, NT - N)] = jnp.zeros((NT - N, D), jnp.float32)


def _readout_body(p_ref, g_ref, pdeg_ref, b3_ref, wr1_ref, br1_ref, wr2_ref,
                  br2_ref, batch_ref, o_ref):
    dinv = _dinv_from(pdeg_ref)
    h = dinv * (p_ref[0][:N] + p_ref[1][:N] + g_ref[:N]) + b3_ref[...]
    t = jnp.maximum(
        jnp.dot(h, wr1_ref[...], preferred_element_type=jnp.float32) + br1_ref[...],
        0.0,
    )
    r = jnp.dot(t, wr2_ref[...], preferred_element_type=jnp.float32) + br2_ref[...]
    onehot = (batch_ref[...] == lax.broadcasted_iota(jnp.int32, (N, NG), 1)
              ).astype(jnp.float32)
    dn = (((0,), (0,)), ((), ()))
    sums = lax.dot_general(onehot, r, dn, preferred_element_type=jnp.float32)
    counts = lax.dot_general(onehot, jnp.ones((N, 1), jnp.float32), dn,
                             preferred_element_type=jnp.float32)
    o_ref[...] = sums / jnp.maximum(counts, 1.0)


def kernel(x, edge_index, batch, W1, b1, W2, b2, W3, b3, Wr1, br1, Wr2, br2):
    src = edge_index[0].astype(jnp.int32)
    dst = edge_index[1].astype(jnp.int32)
    pad = EPAD - E
    srcl = jnp.concatenate([src, jnp.full((pad,), N, jnp.int32)]).reshape(NW, K, CH)
    pad_dst = jnp.arange(pad, dtype=jnp.int32) % N
    dstl = jnp.concatenate([dst, pad_dst]).reshape(NW, K, CH)
    sdl = jnp.stack([srcl, dstl], axis=2)
    zf = jnp.zeros((RPT, D), jnp.float32)

    onest = jnp.concatenate([jnp.ones((N, D), jnp.float32),
                             jnp.zeros((NT - N, D), jnp.float32)])
    pdeg = _edge_scatter(onest, sdl, zf)

    g1 = pl.pallas_call(
        _k1_body, out_shape=jax.ShapeDtypeStruct((NT, D), jnp.float32),
    )(x, W1, pdeg)
    p1 = _edge_scatter(g1, sdl, zf)

    mid = pl.pallas_call(
        _mid_body, out_shape=jax.ShapeDtypeStruct((NT, D), jnp.float32),
    )
    g2 = mid(p1, g1, pdeg, W2, b1.reshape(1, D))
    p2 = _edge_scatter(g2, sdl, zf)

    g3 = mid(p2, g2, pdeg, W3, b2.reshape(1, D))
    p3 = _edge_scatter(g3, sdl, zf)

    out = pl.pallas_call(
        _readout_body, out_shape=jax.ShapeDtypeStruct((NG, 1), jnp.float32),
    )(p3, g3, pdeg, b3.reshape(1, D), Wr1, br1.reshape(1, D // 2), Wr2,
      br2.reshape(1, 1), batch.astype(jnp.int32).reshape(N, 1))
    return out


# speedup vs baseline: 12.9817x; 1.1960x over previous
import functools

import jax
import jax.numpy as jnp
from jax import lax
from jax.experimental import pallas as pl
from jax.experimental.pallas import tpu as pltpu
from jax.experimental.pallas import tpu_sc as plsc

N = 10000
E = 320000
D = 128
NG = 64
NC = 2
NS = 16
NW = NC * NS
CH = 128
K = -(-E // (NW * CH))
EPAD = NW * CH * K
RPT = 632
NACC = NS * RPT
NT = N + 8


def _sc_mesh():
    return plsc.VectorSubcoreMesh(
        core_axis_name="c", subcore_axis_name="s", num_cores=NC, num_subcores=NS
    )


def _scatter_body(g_hbm, sdl, zrows, out_hbm, sd0, sd1, rows0, rows1, acc,
                  isem0, isem1, gsem0, gsem1):
    c = lax.axis_index("c")
    s = lax.axis_index("s")
    w = c * NS + s
    pltpu.sync_copy(zrows, acc.at[pl.ds(s * RPT, RPT)])
    plsc.subcore_barrier()

    pltpu.sync_copy(sdl.at[w, 0], sd0)
    pltpu.async_copy(g_hbm.at[sd0.at[0]], rows0, gsem0)
    pltpu.async_copy(sdl.at[w, 1], sd1, isem1)

    def pair(i, carry):
        k0 = i * 2
        pltpu.make_async_copy(sdl.at[w, k0 + 1], sd1, isem1).wait()
        pltpu.async_copy(g_hbm.at[sd1.at[0]], rows1, gsem1)
        pltpu.make_async_copy(g_hbm.at[sd0.at[0]], rows0, gsem0).wait()
        pltpu.sync_copy(rows0, acc.at[sd0.at[1]], add=True)
        pltpu.async_copy(sdl.at[w, k0 + 2], sd0, isem0)
        pltpu.make_async_copy(sdl.at[w, k0 + 2], sd0, isem0).wait()
        pltpu.async_copy(g_hbm.at[sd0.at[0]], rows0, gsem0)
        pltpu.make_async_copy(g_hbm.at[sd1.at[0]], rows1, gsem1).wait()
        pltpu.sync_copy(rows1, acc.at[sd1.at[1]], add=True)
        pltpu.async_copy(sdl.at[w, k0 + 3], sd1, isem1)
        return carry

    lax.fori_loop(0, (K - 3) // 2, pair, 0)
    pltpu.make_async_copy(sdl.at[w, K - 2], sd1, isem1).wait()
    pltpu.async_copy(g_hbm.at[sd1.at[0]], rows1, gsem1)
    pltpu.make_async_copy(g_hbm.at[sd0.at[0]], rows0, gsem0).wait()
    pltpu.sync_copy(rows0, acc.at[sd0.at[1]], add=True)
    pltpu.sync_copy(sdl.at[w, K - 1], sd0)
    pltpu.async_copy(g_hbm.at[sd0.at[0]], rows0, gsem0)
    pltpu.make_async_copy(g_hbm.at[sd1.at[0]], rows1, gsem1).wait()
    pltpu.sync_copy(rows1, acc.at[sd1.at[1]], add=True)
    pltpu.make_async_copy(g_hbm.at[sd0.at[0]], rows0, gsem0).wait()
    pltpu.sync_copy(rows0, acc.at[sd0.at[1]], add=True)
    plsc.subcore_barrier()
    pltpu.sync_copy(acc.at[pl.ds(s * RPT, RPT)], out_hbm.at[c, pl.ds(s * RPT, RPT)])


def _edge_scatter(g, sdl, zrows):
    return pl.kernel(
        _scatter_body,
        out_type=jax.ShapeDtypeStruct((NC, NACC, D), jnp.float32),
        mesh=_sc_mesh(),
        scratch_types=[
            pltpu.VMEM((2, CH), jnp.int32),
            pltpu.VMEM((2, CH), jnp.int32),
            pltpu.VMEM((CH, D), jnp.float32),
            pltpu.VMEM((CH, D), jnp.float32),
            pltpu.VMEM_SHARED((NACC, D), jnp.float32),
            pltpu.SemaphoreType.DMA,
            pltpu.SemaphoreType.DMA,
            pltpu.SemaphoreType.DMA,
            pltpu.SemaphoreType.DMA,
        ],
    )(g, sdl, zrows)


def _deg_body(ones_hbm, sdl, zrows, out_hbm, sd0, sd1, rows0, acc, isem0, isem1):
    c = lax.axis_index("c")
    s = lax.axis_index("s")
    w = c * NS + s
    pltpu.sync_copy(zrows, acc.at[pl.ds(s * RPT, RPT)])
    pltpu.sync_copy(ones_hbm.at[pl.ds(0, CH)], rows0)
    plsc.subcore_barrier()

    pltpu.sync_copy(sdl.at[w, 0], sd0)
    pltpu.async_copy(sdl.at[w, 1], sd1, isem1)

    def pair(i, carry):
        k0 = i * 2
        pltpu.sync_copy(rows0, acc.at[sd0.at[1]], add=True)
        pltpu.async_copy(sdl.at[w, k0 + 2], sd0, isem0)
        pltpu.make_async_copy(sdl.at[w, k0 + 1], sd1, isem1).wait()
        pltpu.sync_copy(rows0, acc.at[sd1.at[1]], add=True)
        pltpu.async_copy(sdl.at[w, k0 + 3], sd1, isem1)
        pltpu.make_async_copy(sdl.at[w, k0 + 2], sd0, isem0).wait()
        return carry

    lax.fori_loop(0, (K - 3) // 2, pair, 0)
    pltpu.sync_copy(rows0, acc.at[sd0.at[1]], add=True)
    pltpu.make_async_copy(sdl.at[w, K - 2], sd1, isem1).wait()
    pltpu.sync_copy(rows0, acc.at[sd1.at[1]], add=True)
    pltpu.sync_copy(sdl.at[w, K - 1], sd0)
    pltpu.sync_copy(rows0, acc.at[sd0.at[1]], add=True)
    plsc.subcore_barrier()
    pltpu.sync_copy(acc.at[pl.ds(s * RPT, RPT)], out_hbm.at[c, pl.ds(s * RPT, RPT)])


def _edge_degree(ones, sdl, zrows):
    return pl.kernel(
        _deg_body,
        out_type=jax.ShapeDtypeStruct((NC, NACC, D), jnp.float32),
        mesh=_sc_mesh(),
        scratch_types=[
            pltpu.VMEM((2, CH), jnp.int32),
            pltpu.VMEM((2, CH), jnp.int32),
            pltpu.VMEM((CH, D), jnp.float32),
            pltpu.VMEM_SHARED((NACC, D), jnp.float32),
            pltpu.SemaphoreType.DMA,
            pltpu.SemaphoreType.DMA,
        ],
    )(ones, sdl, zrows)


PAD = EPAD - E


def _dinv_from(pdeg_ref):
    d = pdeg_ref[0][:N, 0:1] + pdeg_ref[1][:N, 0:1]
    rid = lax.broadcasted_iota(jnp.int32, (N, 1), 0)
    d = d - jnp.where(rid < PAD, 1.0, 0.0)
    return lax.rsqrt(d + 1.0)


def _k1_body(x_ref, w_ref, pdeg_ref, g_ref):
    dinv = _dinv_from(pdeg_ref)
    h = jnp.dot(x_ref[...], w_ref[...], preferred_element_type=jnp.float32)
    g_ref[pl.ds(0, N)] = h * dinv
    g_ref[pl.ds(N, NT - N)] = jnp.zeros((NT - N, D), jnp.float32)


def _mid_body(p_ref, g_ref, pdeg_ref, w_ref, b_ref, o_ref):
    dinv = _dinv_from(pdeg_ref)
    h = dinv * (p_ref[0][:N] + p_ref[1][:N] + g_ref[:N]) + b_ref[...]
    a = jnp.maximum(h, 0.0)
    o_ref[pl.ds(0, N)] = dinv * jnp.dot(a, w_ref[...], preferred_element_type=jnp.float32)
    o_ref[pl.ds(N, NT - N)] = jnp.zeros((NT - N, D), jnp.float32)


def _readout_body(p_ref, g_ref, pdeg_ref, b3_ref, wr1_ref, br1_ref, wr2_ref,
                  br2_ref, batch_ref, o_ref):
    dinv = _dinv_from(pdeg_ref)
    h = dinv * (p_ref[0][:N] + p_ref[1][:N] + g_ref[:N]) + b3_ref[...]
    t = jnp.maximum(
        jnp.dot(h, wr1_ref[...], preferred_element_type=jnp.float32) + br1_ref[...],
        0.0,
    )
    r = jnp.dot(t, wr2_ref[...], preferred_element_type=jnp.float32) + br2_ref[...]
    onehot = (batch_ref[...] == lax.broadcasted_iota(jnp.int32, (N, NG), 1)
              ).astype(jnp.float32)
    dn = (((0,), (0,)), ((), ()))
    sums = lax.dot_general(onehot, r, dn, preferred_element_type=jnp.float32)
    counts = lax.dot_general(onehot, jnp.ones((N, 1), jnp.float32), dn,
                             preferred_element_type=jnp.float32)
    o_ref[...] = sums / jnp.maximum(counts, 1.0)


def kernel(x, edge_index, batch, W1, b1, W2, b2, W3, b3, Wr1, br1, Wr2, br2):
    src = edge_index[0].astype(jnp.int32)
    dst = edge_index[1].astype(jnp.int32)
    pad = EPAD - E
    srcl = jnp.concatenate([src, jnp.full((pad,), N, jnp.int32)]).reshape(NW, K, CH)
    pad_dst = jnp.arange(pad, dtype=jnp.int32) % N
    dstl = jnp.concatenate([dst, pad_dst]).reshape(NW, K, CH)
    sdl = jnp.stack([srcl, dstl], axis=2)
    zf = jnp.zeros((RPT, D), jnp.float32)

    onest = jnp.concatenate([jnp.ones((N, D), jnp.float32),
                             jnp.zeros((NT - N, D), jnp.float32)])
    pdeg = _edge_degree(onest, sdl, zf)

    g1 = pl.pallas_call(
        _k1_body, out_shape=jax.ShapeDtypeStruct((NT, D), jnp.float32),
    )(x, W1, pdeg)
    p1 = _edge_scatter(g1, sdl, zf)

    mid = pl.pallas_call(
        _mid_body, out_shape=jax.ShapeDtypeStruct((NT, D), jnp.float32),
    )
    g2 = mid(p1, g1, pdeg, W2, b1.reshape(1, D))
    p2 = _edge_scatter(g2, sdl, zf)

    g3 = mid(p2, g2, pdeg, W3, b2.reshape(1, D))
    p3 = _edge_scatter(g3, sdl, zf)

    out = pl.pallas_call(
        _readout_body, out_shape=jax.ShapeDtypeStruct((NG, 1), jnp.float32),
    )(p3, g3, pdeg, b3.reshape(1, D), Wr1, br1.reshape(1, D // 2), Wr2,
      br2.reshape(1, 1), batch.astype(jnp.int32).reshape(N, 1))
    return out
